# Initial kernel scaffold; baseline (speedup 1.0000x reference)
import jax
import jax.numpy as jnp
from jax.experimental import pallas as pl


def kernel(guidance, ini_depth, sparse_depth, We1, be1, Wa1, ba1, We2, be2, Wa2, ba2, We3, be3, Wa3, ba3):
    raise NotImplementedError("write your pallas kernel here")


# speedup vs baseline: 13.9677x; 13.9677x over previous
import functools

import jax
import jax.numpy as jnp
from jax import lax
from jax.experimental import pallas as pl
from jax.experimental.pallas import tpu as pltpu
from jax.experimental.pallas import tpu_sc as plsc
import numpy as np

N_REAL = 76 * 102
N_PAD = 61 * 128
K = 16
RB = 128
N_BLOCKS = N_PAD // RB


def _camera_np():
    xx, yy = np.meshgrid(np.arange(0, 102, 1), np.arange(0, 76, 1))
    fx_d = 582.6244816773795 / 2.0
    fy_d = 582.6910327098864 / 2.0
    cx_d = 313.0447587080473 / 2.0
    cy_d = 238.44389626620386 / 2.0
    x_3d = ((xx - cx_d) / fx_d).astype(np.float32)
    y_3d = ((yy - cy_d) / fy_d).astype(np.float32)
    return jnp.asarray(x_3d), jnp.asarray(y_3d)


def _knn_body(x_ref, xt_ref, idx_ref):
    xb = x_ref[...]
    xt = xt_ref[...]
    inner = lax.dot_general(xb.astype(jnp.bfloat16), xt.astype(jnp.bfloat16),
                            (((1,), (0,)), ((), ())),
                            preferred_element_type=jnp.float32)
    sq = jnp.sum(xb * xb, axis=1, keepdims=True)
    sqt = jnp.sum(xt * xt, axis=0, keepdims=True)
    d = (sq + (-2.0) * inner) + sqt
    col = lax.broadcasted_iota(jnp.int32, d.shape, 1)
    d = jnp.where(col < N_REAL, d, jnp.inf)
    cols_out = []
    for _ in range(K):
        m = jnp.min(d, axis=1, keepdims=True)
        cand = jnp.where(d == m, col, jnp.int32(2**31 - 1))
        am = jnp.min(cand, axis=1, keepdims=True)
        cols_out.append(am)
        d = jnp.where(col == am, jnp.inf, d)
    idx_ref[...] = jnp.concatenate(cols_out, axis=1)


def _knn(x, xt):
    c = x.shape[1]
    return pl.pallas_call(
        _knn_body,
        grid=(N_BLOCKS,),
        in_specs=[
            pl.BlockSpec((RB, c), lambda i: (i, 0)),
            pl.BlockSpec((c, N_PAD), lambda i: (0, 0)),
        ],
        out_specs=pl.BlockSpec((RB, K), lambda i: (i, 0)),
        out_shape=jax.ShapeDtypeStruct((N_PAD, K), jnp.int32),
    )(x, xt)


def _proj_body(x_ref, w_ref, b_ref, o_ref):
    o_ref[...] = lax.dot_general(x_ref[...], w_ref[...], (((1,), (0,)), ((), ())),
                                 preferred_element_type=jnp.float32) + b_ref[...]


def _proj(x, w, brow):
    c, d4 = w.shape
    rb = 976
    return pl.pallas_call(
        _proj_body,
        grid=(N_PAD // rb,),
        in_specs=[
            pl.BlockSpec((rb, c), lambda i: (i, 0)),
            pl.BlockSpec((c, d4), lambda i: (0, 0)),
            pl.BlockSpec((1, d4), lambda i: (0, 0)),
        ],
        out_specs=pl.BlockSpec((rb, d4), lambda i: (i, 0)),
        out_shape=jax.ShapeDtypeStruct((N_PAD, d4), jnp.float32),
    )(x, w, brow)


def _sc_gather(table, idxflat):
    e_total = idxflat.shape[0]
    d = table.shape[1]
    win = 128
    idx2 = idxflat.reshape(1, e_total)
    mesh = plsc.VectorSubcoreMesh(core_axis_name="core", subcore_axis_name="subcore")

    @functools.partial(
        pl.kernel,
        out_type=jax.ShapeDtypeStruct((e_total, d), jnp.float32),
        mesh=mesh,
        compiler_params=pltpu.CompilerParams(use_tc_tiling_on_sc=False),
    )
    def k(x_hbm, i_hbm, o_hbm):
        def body(i_vmem, o_vmem):
            pltpu.sync_copy(x_hbm.at[i_vmem.at[0]], o_vmem)

        pltpu.emit_pipeline(
            body,
            grid=(e_total // win,),
            in_specs=[pl.BlockSpec((1, win), index_map=lambda i: (0, i))],
            out_specs=[pl.BlockSpec((win, d), index_map=lambda i: (i, 0))],
            core_axis_name=("core", "subcore"),
            dimension_semantics=(pltpu.PARALLEL,),
        )(i_hbm, o_hbm)

    return k(table, idx2)


def _combine_body(op, g_ref, ue_ref, ua_ref, o_ref):
    ue = ue_ref[...]
    ua = ua_ref[...]
    eks, aks = [], []
    for k in range(K):
        gk = g_ref[k]
        eks.append(gk[:, :op] + ue)
        aks.append(gk[:, op:] + ua)
    m = aks[0]
    for k in range(1, K):
        m = jnp.maximum(m, aks[k])
    s = jnp.zeros_like(m)
    acc = jnp.zeros_like(m)
    for k in range(K):
        w = jnp.exp(aks[k] - m)
        s = s + w
        acc = acc + w * eks[k]
    o_ref[...] = acc / s


def _combine(g, ue, ua):
    op = ue.shape[1]
    return pl.pallas_call(
        functools.partial(_combine_body, op),
        grid=(N_BLOCKS,),
        in_specs=[
            pl.BlockSpec((K, RB, 2 * op), lambda i: (0, i, 0)),
            pl.BlockSpec((RB, op), lambda i: (i, 0)),
            pl.BlockSpec((RB, op), lambda i: (i, 0)),
        ],
        out_specs=pl.BlockSpec((RB, op), lambda i: (i, 0)),
        out_shape=jax.ShapeDtypeStruct((N_PAD, op), jnp.float32),
    )(g, ue, ua)


def _combine_exact_body(c, g_ref, x_ref, we_ref, wa_ref, be_ref, ba_ref, o_ref):
    xi = x_ref[...][:, :c]
    xib = xi.astype(jnp.bfloat16)
    cats = []
    for k in range(K):
        xj = g_ref[k][:, :c]
        cats.append(jnp.concatenate([xib, (xj - xi).astype(jnp.bfloat16)], axis=1))
    cat = jnp.concatenate(cats, axis=0)
    dn = (((1,), (0,)), ((), ()))
    e = lax.dot_general(cat, we_ref[...], dn,
                        preferred_element_type=jnp.float32) + be_ref[...]
    a = lax.dot_general(cat, wa_ref[...], dn,
                        preferred_element_type=jnp.float32) + ba_ref[...]
    eks = [e[k * RB:(k + 1) * RB] for k in range(K)]
    aks = [a[k * RB:(k + 1) * RB] for k in range(K)]
    m = aks[0]
    for k in range(1, K):
        m = jnp.maximum(m, aks[k])
    ws = [jnp.exp(ak - m) for ak in aks]
    s = ws[0]
    for k in range(1, K):
        s = s + ws[k]
    acc = eks[0] * (ws[0] / s)
    for k in range(1, K):
        acc = acc + eks[k] * (ws[k] / s)
    o_ref[...] = acc


def _attn_exact(x, idx, we, be, wa, ba):
    cp = x.shape[1]
    o, c2 = we.shape
    c = c2 // 2

    wet = jnp.concatenate([we[:, :c].T, we[:, c:].T], axis=0).astype(jnp.bfloat16)
    wat = jnp.concatenate([wa[:, :c].T, wa[:, c:].T], axis=0).astype(jnp.bfloat16)
    idxflat = idx.T.reshape(-1)
    g = _sc_gather(x, idxflat).reshape(K, N_PAD, cp)
    return pl.pallas_call(
        functools.partial(_combine_exact_body, c),
        grid=(N_BLOCKS,),
        in_specs=[
            pl.BlockSpec((K, RB, cp), lambda i: (0, i, 0)),
            pl.BlockSpec((RB, cp), lambda i: (i, 0)),
            pl.BlockSpec((2 * c, o), lambda i: (0, 0)),
            pl.BlockSpec((2 * c, o), lambda i: (0, 0)),
            pl.BlockSpec((1, o), lambda i: (0, 0)),
            pl.BlockSpec((1, o), lambda i: (0, 0)),
        ],
        out_specs=pl.BlockSpec((RB, o), lambda i: (i, 0)),
        out_shape=jax.ShapeDtypeStruct((N_PAD, o), jnp.float32),
    )(g, x, wet, wat, be[None, :], ba[None, :])


def _attn_layer(x, idx, we, be, wa, ba, op):
    c = x.shape[1]
    o = we.shape[0]
    a_e, b_e = we[:, :c], we[:, c:]
    a_a, b_a = wa[:, :c], wa[:, c:]

    def padw(mat):
        return jnp.pad(mat.T, ((0, 0), (0, op - o)))

    w4 = jnp.concatenate(
        [padw(a_e - b_e), padw(a_a - b_a), padw(b_e), padw(b_a)], axis=1)
    brow = jnp.concatenate(
        [jnp.pad(be, (0, op - o)), jnp.pad(ba, (0, op - o)),
         jnp.zeros((2 * op,), jnp.float32)])[None, :]
    p = _proj(x, w4, brow)
    ue, ua = p[:, :op], p[:, op:2 * op]
    table = p[:, 2 * op:]
    idxflat = idx.T.reshape(-1)
    g = _sc_gather(table, idxflat).reshape(K, N_PAD, 2 * op)
    return _combine(g, ue, ua)


def _build_feats(guidance, ini_depth, sparse_depth):
    x_3d, y_3d = _camera_np()
    mask = jnp.sign(sparse_depth)
    ini = (1.0 - mask) * ini_depth + mask * sparse_depth
    ini_s = ini[0, 0, 1::3, 0::3].astype(jnp.bfloat16).astype(jnp.float32)
    x3 = x_3d * ini_s * 3.0
    y3 = y_3d * ini_s * 3.0
    loc = jnp.stack([x3, y3, ini_s], axis=-1).reshape(N_REAL, 3)
    gpad = jnp.pad(guidance[0], ((0, 0), (3, 3), (4, 4)))
    chans = [
        lax.slice(gpad, (t, t // 9, t % 9),
                  (t + 1, t // 9 + 226, t % 9 + 304), (1, 3, 3))[0]
        for t in range(81)
    ]
    g_feat = jnp.stack(chans, axis=-1).reshape(N_REAL, 81)
    g_feat = g_feat.astype(jnp.bfloat16).astype(jnp.float32)
    locp = jnp.pad(loc, ((0, N_PAD - N_REAL), (0, 5)))
    xg = jnp.pad(g_feat, ((0, N_PAD - N_REAL), (0, 15)))
    return locp, xg


def kernel(guidance, ini_depth, sparse_depth, We1, be1, Wa1, ba1,
           We2, be2, Wa2, ba2, We3, be3, Wa3, ba3):
    locp, xg = _build_feats(guidance, ini_depth, sparse_depth)

    i1 = _knn(locp, locp.T)
    h = _attn_exact(xg, i1, We1, be1, Wa1, ba1)
    i2 = _knn(h, h.T)
    h = _attn_exact(h, i2, We2, be2, Wa2, ba2)
    i3 = _knn(h, h.T)
    h = _attn_layer(h, i3, We3, be3, Wa3, ba3, 16)

    h9 = h[:N_REAL, :9].reshape(76, 102, 3, 3)
    out = jnp.transpose(h9, (0, 2, 1, 3)).reshape(228, 306)
    return out[None, None, :, 1:305]


# speedup vs baseline: 15.8278x; 1.1332x over previous
import functools

import jax
import jax.numpy as jnp
from jax import lax
from jax.experimental import pallas as pl
from jax.experimental.pallas import tpu as pltpu
from jax.experimental.pallas import tpu_sc as plsc
import numpy as np

N_REAL = 76 * 102
N_PAD = 61 * 128
K = 16
RB = 128
N_BLOCKS = N_PAD // RB


def _camera_np():
    xx, yy = np.meshgrid(np.arange(0, 102, 1), np.arange(0, 76, 1))
    fx_d = 582.6244816773795 / 2.0
    fy_d = 582.6910327098864 / 2.0
    cx_d = 313.0447587080473 / 2.0
    cy_d = 238.44389626620386 / 2.0
    x_3d = ((xx - cx_d) / fx_d).astype(np.float32)
    y_3d = ((yy - cy_d) / fy_d).astype(np.float32)
    return jnp.asarray(x_3d), jnp.asarray(y_3d)


def _knn_body(x_ref, xt_ref, idx_ref):
    xb = x_ref[...]
    xt = xt_ref[...]
    inner = lax.dot_general(xb.astype(jnp.bfloat16), xt.astype(jnp.bfloat16),
                            (((1,), (0,)), ((), ())),
                            preferred_element_type=jnp.float32)
    sq = jnp.sum(xb * xb, axis=1, keepdims=True)
    sqt = jnp.sum(xt * xt, axis=0, keepdims=True)
    d = (sq + (-2.0) * inner) + sqt
    colf = lax.broadcasted_iota(jnp.int32, d.shape, 1).astype(jnp.float32)
    d = jnp.where(colf < float(N_REAL), d, jnp.inf)
    big = jnp.float32(3e9)
    cols_out = []
    for _ in range(K):
        m = jnp.min(d, axis=1, keepdims=True)
        cand = jnp.where(d == m, colf, big)
        am = jnp.min(cand, axis=1, keepdims=True)
        cols_out.append(am)
        d = jnp.where(colf == am, jnp.inf, d)
    idx_ref[...] = jnp.concatenate(cols_out, axis=1).astype(jnp.int32)


def _knn(x, xt):
    c = x.shape[1]
    return pl.pallas_call(
        _knn_body,
        grid=(N_BLOCKS,),
        in_specs=[
            pl.BlockSpec((RB, c), lambda i: (i, 0)),
            pl.BlockSpec((c, N_PAD), lambda i: (0, 0)),
        ],
        out_specs=pl.BlockSpec((RB, K), lambda i: (i, 0)),
        out_shape=jax.ShapeDtypeStruct((N_PAD, K), jnp.int32),
    )(x, xt)


def _proj_body(x_ref, w_ref, b_ref, o_ref):
    o_ref[...] = lax.dot_general(x_ref[...], w_ref[...], (((1,), (0,)), ((), ())),
                                 preferred_element_type=jnp.float32) + b_ref[...]


def _proj(x, w, brow):
    c, d4 = w.shape
    rb = 976
    return pl.pallas_call(
        _proj_body,
        grid=(N_PAD // rb,),
        in_specs=[
            pl.BlockSpec((rb, c), lambda i: (i, 0)),
            pl.BlockSpec((c, d4), lambda i: (0, 0)),
            pl.BlockSpec((1, d4), lambda i: (0, 0)),
        ],
        out_specs=pl.BlockSpec((rb, d4), lambda i: (i, 0)),
        out_shape=jax.ShapeDtypeStruct((N_PAD, d4), jnp.float32),
    )(x, w, brow)


def _sc_gather(table, idxflat):
    e_total = idxflat.shape[0]
    d = table.shape[1]
    win = 128
    idx2 = idxflat.reshape(1, e_total)
    mesh = plsc.VectorSubcoreMesh(core_axis_name="core", subcore_axis_name="subcore")

    @functools.partial(
        pl.kernel,
        out_type=jax.ShapeDtypeStruct((e_total, d), jnp.float32),
        mesh=mesh,
        compiler_params=pltpu.CompilerParams(use_tc_tiling_on_sc=False),
    )
    def k(x_hbm, i_hbm, o_hbm):
        def body(i_vmem, o_vmem):
            pltpu.sync_copy(x_hbm.at[i_vmem.at[0]], o_vmem)

        pltpu.emit_pipeline(
            body,
            grid=(e_total // win,),
            in_specs=[pl.BlockSpec((1, win), index_map=lambda i: (0, i))],
            out_specs=[pl.BlockSpec((win, d), index_map=lambda i: (i, 0))],
            core_axis_name=("core", "subcore"),
            dimension_semantics=(pltpu.PARALLEL,),
        )(i_hbm, o_hbm)

    return k(table, idx2)


def _combine_body(op, g_ref, ue_ref, ua_ref, o_ref):
    ue = ue_ref[...]
    ua = ua_ref[...]
    eks, aks = [], []
    for k in range(K):
        gk = g_ref[k]
        eks.append(gk[:, :op] + ue)
        aks.append(gk[:, op:] + ua)
    m = aks[0]
    for k in range(1, K):
        m = jnp.maximum(m, aks[k])
    s = jnp.zeros_like(m)
    acc = jnp.zeros_like(m)
    for k in range(K):
        w = jnp.exp(aks[k] - m)
        s = s + w
        acc = acc + w * eks[k]
    o_ref[...] = acc / s


def _combine(g, ue, ua):
    op = ue.shape[1]
    return pl.pallas_call(
        functools.partial(_combine_body, op),
        grid=(N_BLOCKS,),
        in_specs=[
            pl.BlockSpec((K, RB, 2 * op), lambda i: (0, i, 0)),
            pl.BlockSpec((RB, op), lambda i: (i, 0)),
            pl.BlockSpec((RB, op), lambda i: (i, 0)),
        ],
        out_specs=pl.BlockSpec((RB, op), lambda i: (i, 0)),
        out_shape=jax.ShapeDtypeStruct((N_PAD, op), jnp.float32),
    )(g, ue, ua)


def _combine_exact_body(c, g_ref, x_ref, we_ref, wa_ref, be_ref, ba_ref, o_ref):
    xi = x_ref[...][:, :c]
    xib = xi.astype(jnp.bfloat16)
    cats = []
    for k in range(K):
        xj = g_ref[k][:, :c]
        cats.append(jnp.concatenate([xib, (xj - xi).astype(jnp.bfloat16)], axis=1))
    cat = jnp.concatenate(cats, axis=0)
    dn = (((1,), (0,)), ((), ()))
    e = lax.dot_general(cat, we_ref[...], dn,
                        preferred_element_type=jnp.float32) + be_ref[...]
    a = lax.dot_general(cat, wa_ref[...], dn,
                        preferred_element_type=jnp.float32) + ba_ref[...]
    eks = [e[k * RB:(k + 1) * RB] for k in range(K)]
    aks = [a[k * RB:(k + 1) * RB] for k in range(K)]
    m = aks[0]
    for k in range(1, K):
        m = jnp.maximum(m, aks[k])
    ws = [jnp.exp(ak - m) for ak in aks]
    s = ws[0]
    for k in range(1, K):
        s = s + ws[k]
    acc = eks[0] * (ws[0] / s)
    for k in range(1, K):
        acc = acc + eks[k] * (ws[k] / s)
    o_ref[...] = acc


def _attn_exact(x, idx, we, be, wa, ba):
    cp = x.shape[1]
    o, c2 = we.shape
    c = c2 // 2

    wet = jnp.concatenate([we[:, :c].T, we[:, c:].T], axis=0).astype(jnp.bfloat16)
    wat = jnp.concatenate([wa[:, :c].T, wa[:, c:].T], axis=0).astype(jnp.bfloat16)
    idxflat = idx.T.reshape(-1)
    g = _sc_gather(x, idxflat).reshape(K, N_PAD, cp)
    return pl.pallas_call(
        functools.partial(_combine_exact_body, c),
        grid=(N_BLOCKS,),
        in_specs=[
            pl.BlockSpec((K, RB, cp), lambda i: (0, i, 0)),
            pl.BlockSpec((RB, cp), lambda i: (i, 0)),
            pl.BlockSpec((2 * c, o), lambda i: (0, 0)),
            pl.BlockSpec((2 * c, o), lambda i: (0, 0)),
            pl.BlockSpec((1, o), lambda i: (0, 0)),
            pl.BlockSpec((1, o), lambda i: (0, 0)),
        ],
        out_specs=pl.BlockSpec((RB, o), lambda i: (i, 0)),
        out_shape=jax.ShapeDtypeStruct((N_PAD, o), jnp.float32),
    )(g, x, wet, wat, be[None, :], ba[None, :])


def _attn_layer(x, idx, we, be, wa, ba, op):
    c = x.shape[1]
    o = we.shape[0]
    a_e, b_e = we[:, :c], we[:, c:]
    a_a, b_a = wa[:, :c], wa[:, c:]

    def padw(mat):
        return jnp.pad(mat.T, ((0, 0), (0, op - o)))

    w4 = jnp.concatenate(
        [padw(a_e - b_e), padw(a_a - b_a), padw(b_e), padw(b_a)], axis=1)
    brow = jnp.concatenate(
        [jnp.pad(be, (0, op - o)), jnp.pad(ba, (0, op - o)),
         jnp.zeros((2 * op,), jnp.float32)])[None, :]
    p = _proj(x, w4, brow)
    ue, ua = p[:, :op], p[:, op:2 * op]
    table = p[:, 2 * op:]
    idxflat = idx.T.reshape(-1)
    g = _sc_gather(table, idxflat).reshape(K, N_PAD, 2 * op)
    return _combine(g, ue, ua)


def _build_feats(guidance, ini_depth, sparse_depth):
    x_3d, y_3d = _camera_np()
    mask = jnp.sign(sparse_depth)
    ini = (1.0 - mask) * ini_depth + mask * sparse_depth
    ini_s = ini[0, 0, 1::3, 0::3].astype(jnp.bfloat16).astype(jnp.float32)
    x3 = x_3d * ini_s * 3.0
    y3 = y_3d * ini_s * 3.0
    loc = jnp.stack([x3, y3, ini_s], axis=-1).reshape(N_REAL, 3)
    gpad = jnp.pad(guidance[0], ((0, 0), (3, 3), (4, 4)))
    chans = [
        lax.slice(gpad, (t, t // 9, t % 9),
                  (t + 1, t // 9 + 226, t % 9 + 304), (1, 3, 3))[0]
        for t in range(81)
    ]
    g_feat = jnp.stack(chans, axis=-1).reshape(N_REAL, 81)
    g_feat = g_feat.astype(jnp.bfloat16).astype(jnp.float32)
    locp = jnp.pad(loc, ((0, N_PAD - N_REAL), (0, 5)))
    xg = jnp.pad(g_feat, ((0, N_PAD - N_REAL), (0, 15)))
    return locp, xg


def kernel(guidance, ini_depth, sparse_depth, We1, be1, Wa1, ba1,
           We2, be2, Wa2, ba2, We3, be3, Wa3, ba3):
    locp, xg = _build_feats(guidance, ini_depth, sparse_depth)

    i1 = _knn(locp, locp.T)
    h = _attn_exact(xg, i1, We1, be1, Wa1, ba1)
    i2 = _knn(h, h.T)
    h = _attn_exact(h, i2, We2, be2, Wa2, ba2)
    i3 = _knn(h, h.T)
    h = _attn_layer(h, i3, We3, be3, Wa3, ba3, 16)

    h9 = h[:N_REAL, :9].reshape(76, 102, 3, 3)
    out = jnp.transpose(h9, (0, 2, 1, 3)).reshape(228, 306)
    return out[None, None, :, 1:305]


# speedup vs baseline: 16.1586x; 1.0209x over previous
import functools

import jax
import jax.numpy as jnp
from jax import lax
from jax.experimental import pallas as pl
from jax.experimental.pallas import tpu as pltpu
from jax.experimental.pallas import tpu_sc as plsc
import numpy as np

N_REAL = 76 * 102
N_PAD = 61 * 128
K = 16
RB = 128
N_BLOCKS = N_PAD // RB


def _camera_np():
    xx, yy = np.meshgrid(np.arange(0, 102, 1), np.arange(0, 76, 1))
    fx_d = 582.6244816773795 / 2.0
    fy_d = 582.6910327098864 / 2.0
    cx_d = 313.0447587080473 / 2.0
    cy_d = 238.44389626620386 / 2.0
    x_3d = ((xx - cx_d) / fx_d).astype(np.float32)
    y_3d = ((yy - cy_d) / fy_d).astype(np.float32)
    return jnp.asarray(x_3d), jnp.asarray(y_3d)


def _knn_body(x_ref, xt_ref, idx_ref):
    xb = x_ref[...]
    xt = xt_ref[...]
    inner = lax.dot_general(xb.astype(jnp.bfloat16), xt.astype(jnp.bfloat16),
                            (((1,), (0,)), ((), ())),
                            preferred_element_type=jnp.float32)
    sq = jnp.sum(xb * xb, axis=1, keepdims=True)
    sqt = jnp.sum(xt * xt, axis=0, keepdims=True)
    d = (sq + (-2.0) * inner) + sqt
    colf = lax.broadcasted_iota(jnp.int32, d.shape, 1).astype(jnp.float32)
    d = jnp.where(colf < float(N_REAL), d, jnp.inf)
    big = jnp.float32(3e9)
    cols_out = []
    for _ in range(K):
        m = jnp.min(d, axis=1, keepdims=True)
        cand = jnp.where(d == m, colf, big)
        am = jnp.min(cand, axis=1, keepdims=True)
        cols_out.append(am)
        d = jnp.where(colf == am, jnp.inf, d)
    idx_ref[...] = jnp.concatenate(cols_out, axis=1).astype(jnp.int32)


N_TOP = 31 * RB
N_BOT = N_PAD - N_TOP


def _knn(xpart, xt):
    n, c = xpart.shape
    return pl.pallas_call(
        _knn_body,
        grid=(n // RB,),
        in_specs=[
            pl.BlockSpec((RB, c), lambda i: (i, 0)),
            pl.BlockSpec((c, N_PAD), lambda i: (0, 0)),
        ],
        out_specs=pl.BlockSpec((RB, K), lambda i: (i, 0)),
        out_shape=jax.ShapeDtypeStruct((n, K), jnp.int32),
    )(xpart, xt)


def _proj_body(x_ref, w_ref, b_ref, o_ref):
    o_ref[...] = lax.dot_general(x_ref[...], w_ref[...], (((1,), (0,)), ((), ())),
                                 preferred_element_type=jnp.float32) + b_ref[...]


def _proj(x, w, brow):
    c, d4 = w.shape
    rb = 976
    return pl.pallas_call(
        _proj_body,
        grid=(N_PAD // rb,),
        in_specs=[
            pl.BlockSpec((rb, c), lambda i: (i, 0)),
            pl.BlockSpec((c, d4), lambda i: (0, 0)),
            pl.BlockSpec((1, d4), lambda i: (0, 0)),
        ],
        out_specs=pl.BlockSpec((rb, d4), lambda i: (i, 0)),
        out_shape=jax.ShapeDtypeStruct((N_PAD, d4), jnp.float32),
    )(x, w, brow)


def _sc_gather(table, idxflat):
    e_total = idxflat.shape[0]
    d = table.shape[1]
    win = 128
    idx2 = idxflat.reshape(1, e_total)
    mesh = plsc.VectorSubcoreMesh(core_axis_name="core", subcore_axis_name="subcore")

    @functools.partial(
        pl.kernel,
        out_type=jax.ShapeDtypeStruct((e_total, d), jnp.float32),
        mesh=mesh,
        compiler_params=pltpu.CompilerParams(use_tc_tiling_on_sc=False),
    )
    def k(x_hbm, i_hbm, o_hbm):
        def body(i_vmem, o_vmem):
            pltpu.sync_copy(x_hbm.at[i_vmem.at[0]], o_vmem)

        pltpu.emit_pipeline(
            body,
            grid=(e_total // win,),
            in_specs=[pl.BlockSpec((1, win), index_map=lambda i: (0, i))],
            out_specs=[pl.BlockSpec((win, d), index_map=lambda i: (i, 0))],
            core_axis_name=("core", "subcore"),
            dimension_semantics=(pltpu.PARALLEL,),
        )(i_hbm, o_hbm)

    return k(table, idx2)


def _combine_body(op, g_ref, ue_ref, ua_ref, o_ref):
    ue = ue_ref[...]
    ua = ua_ref[...]
    eks, aks = [], []
    for k in range(K):
        gk = g_ref[k]
        eks.append(gk[:, :op] + ue)
        aks.append(gk[:, op:] + ua)
    m = aks[0]
    for k in range(1, K):
        m = jnp.maximum(m, aks[k])
    s = jnp.zeros_like(m)
    acc = jnp.zeros_like(m)
    for k in range(K):
        w = jnp.exp(aks[k] - m)
        s = s + w
        acc = acc + w * eks[k]
    o_ref[...] = acc / s


def _combine(g, ue, ua):
    op = ue.shape[1]
    return pl.pallas_call(
        functools.partial(_combine_body, op),
        grid=(N_BLOCKS,),
        in_specs=[
            pl.BlockSpec((K, RB, 2 * op), lambda i: (0, i, 0)),
            pl.BlockSpec((RB, op), lambda i: (i, 0)),
            pl.BlockSpec((RB, op), lambda i: (i, 0)),
        ],
        out_specs=pl.BlockSpec((RB, op), lambda i: (i, 0)),
        out_shape=jax.ShapeDtypeStruct((N_PAD, op), jnp.float32),
    )(g, ue, ua)


def _combine_exact_body(c, g_ref, x_ref, we_ref, wa_ref, be_ref, ba_ref, o_ref):
    xi = x_ref[...][:, :c]
    xib = xi.astype(jnp.bfloat16)
    cats = []
    for k in range(K):
        xj = g_ref[k][:, :c]
        cats.append(jnp.concatenate([xib, (xj - xi).astype(jnp.bfloat16)], axis=1))
    cat = jnp.concatenate(cats, axis=0)
    dn = (((1,), (0,)), ((), ()))
    e = lax.dot_general(cat, we_ref[...], dn,
                        preferred_element_type=jnp.float32) + be_ref[...]
    a = lax.dot_general(cat, wa_ref[...], dn,
                        preferred_element_type=jnp.float32) + ba_ref[...]
    eks = [e[k * RB:(k + 1) * RB] for k in range(K)]
    aks = [a[k * RB:(k + 1) * RB] for k in range(K)]
    m = aks[0]
    for k in range(1, K):
        m = jnp.maximum(m, aks[k])
    ws = [jnp.exp(ak - m) for ak in aks]
    s = ws[0]
    for k in range(1, K):
        s = s + ws[k]
    acc = eks[0] * (ws[0] / s)
    for k in range(1, K):
        acc = acc + eks[k] * (ws[k] / s)
    o_ref[...] = acc


def _attn_exact_half(x, xpart, idx_part, wet, wat, be, ba, c):
    n, cp = xpart.shape
    o = wet.shape[1]
    g = _sc_gather(x, idx_part.T.reshape(-1)).reshape(K, n, cp)
    return pl.pallas_call(
        functools.partial(_combine_exact_body, c),
        grid=(n // RB,),
        in_specs=[
            pl.BlockSpec((K, RB, cp), lambda i: (0, i, 0)),
            pl.BlockSpec((RB, cp), lambda i: (i, 0)),
            pl.BlockSpec((2 * c, o), lambda i: (0, 0)),
            pl.BlockSpec((2 * c, o), lambda i: (0, 0)),
            pl.BlockSpec((1, o), lambda i: (0, 0)),
            pl.BlockSpec((1, o), lambda i: (0, 0)),
        ],
        out_specs=pl.BlockSpec((RB, o), lambda i: (i, 0)),
        out_shape=jax.ShapeDtypeStruct((n, o), jnp.float32),
    )(g, xpart, wet, wat, be[None, :], ba[None, :])


def _attn_exact_weights(we, wa):
    o, c2 = we.shape
    c = c2 // 2
    wet = jnp.concatenate([we[:, :c].T, we[:, c:].T], axis=0).astype(jnp.bfloat16)
    wat = jnp.concatenate([wa[:, :c].T, wa[:, c:].T], axis=0).astype(jnp.bfloat16)
    return wet, wat, c


def _attn_layer(x, idx, we, be, wa, ba, op):
    c = x.shape[1]
    o = we.shape[0]
    a_e, b_e = we[:, :c], we[:, c:]
    a_a, b_a = wa[:, :c], wa[:, c:]

    def padw(mat):
        return jnp.pad(mat.T, ((0, 0), (0, op - o)))

    w4 = jnp.concatenate(
        [padw(a_e - b_e), padw(a_a - b_a), padw(b_e), padw(b_a)], axis=1)
    brow = jnp.concatenate(
        [jnp.pad(be, (0, op - o)), jnp.pad(ba, (0, op - o)),
         jnp.zeros((2 * op,), jnp.float32)])[None, :]
    p = _proj(x, w4, brow)
    ue, ua = p[:, :op], p[:, op:2 * op]
    table = p[:, 2 * op:]
    idxflat = idx.T.reshape(-1)
    g = _sc_gather(table, idxflat).reshape(K, N_PAD, 2 * op)
    return _combine(g, ue, ua)


def _build_feats(guidance, ini_depth, sparse_depth):
    x_3d, y_3d = _camera_np()
    mask = jnp.sign(sparse_depth)
    ini = (1.0 - mask) * ini_depth + mask * sparse_depth
    ini_s = ini[0, 0, 1::3, 0::3].astype(jnp.bfloat16).astype(jnp.float32)
    x3 = x_3d * ini_s * 3.0
    y3 = y_3d * ini_s * 3.0
    loc = jnp.stack([x3, y3, ini_s], axis=-1).reshape(N_REAL, 3)
    gpad = jnp.pad(guidance[0], ((0, 0), (3, 3), (4, 4)))
    chans = [
        lax.slice(gpad, (t, t // 9, t % 9),
                  (t + 1, t // 9 + 226, t % 9 + 304), (1, 3, 3))[0]
        for t in range(81)
    ]
    g_feat = jnp.stack(chans, axis=-1).reshape(N_REAL, 81)
    g_feat = g_feat.astype(jnp.bfloat16).astype(jnp.float32)
    locp = jnp.pad(loc, ((0, N_PAD - N_REAL), (0, 5)))
    xg = jnp.pad(g_feat, ((0, N_PAD - N_REAL), (0, 15)))
    return locp, xg


def kernel(guidance, ini_depth, sparse_depth, We1, be1, Wa1, ba1,
           We2, be2, Wa2, ba2, We3, be3, Wa3, ba3):
    locp, xg = _build_feats(guidance, ini_depth, sparse_depth)

    def exact_layer(xk, xf, we, be, wa, ba):
        xt = xk.T
        it = _knn(xk[:N_TOP], xt)
        ib = _knn(xk[N_TOP:], xt)
        wet, wat, c = _attn_exact_weights(we, wa)
        ht = _attn_exact_half(xf, xf[:N_TOP], it, wet, wat, be, ba, c)
        hb = _attn_exact_half(xf, xf[N_TOP:], ib, wet, wat, be, ba, c)
        return jnp.concatenate([ht, hb], axis=0)

    h = exact_layer(locp, xg, We1, be1, Wa1, ba1)
    h = exact_layer(h, h, We2, be2, Wa2, ba2)
    i3 = _knn(h, h.T)
    h = _attn_layer(h, i3, We3, be3, Wa3, ba3, 16)

    h9 = h[:N_REAL, :9].reshape(76, 102, 3, 3)
    out = jnp.transpose(h9, (0, 2, 1, 3)).reshape(228, 306)
    return out[None, None, :, 1:305]
